# baseline (speedup 1.0000x reference)
import functools

import numpy as np

import jax
import jax.numpy as jnp
from jax import lax
from jax.experimental import pallas as pl
from jax.experimental.pallas import tpu as pltpu
from jax.experimental.pallas import tpu_sc as plsc

NC = 2
NS = 16
NW = NC * NS
L = 16
C = 80
CB = 128
C2 = 64


def _zero2d(ref, nrows, ncols):
    z = jnp.zeros((L,), jnp.float32)

    def body(r, carry):
        for jj in range(ncols // L):
            ref[r, pl.ds(jj * L, L)] = z
        return carry

    lax.fori_loop(0, nrows, body, 0)


def _qperm(d):
    q = np.empty((d,), np.int32)
    for g in range(d // 32):
        for t in range(L):
            q[32 * g + t] = 32 * g + 2 * t
            q[32 * g + L + t] = 32 * g + 2 * t + 1
    return jnp.asarray(q)


def _mesh(nc=NC):
    return plsc.VectorSubcoreMesh(core_axis_name="c", subcore_axis_name="s",
                                  num_cores=nc)


def _pad_n(n):
    return ((n + NS * L - 1) // (NS * L)) * (NS * L)


def _make_deg_kernel(n, n_blk):
    np_ = _pad_n(n)
    rows_per_tile = np_ // NS
    blocks_per_w = n_blk // NW
    assert blocks_per_w % 2 == 0

    @functools.partial(
        pl.kernel,
        mesh=_mesh(),
        out_type=jax.ShapeDtypeStruct((NC * 4 * np_,), jnp.float32),
        scratch_types=[
            [pltpu.VMEM_SHARED((np_,), jnp.float32) for _ in range(4)],
            [pltpu.VMEM((2, CB), jnp.int32) for _ in range(2)],
            [pltpu.VMEM((CB,), jnp.float32) for _ in range(2)],
            pltpu.VMEM((CB,), jnp.float32),
            pltpu.VMEM((rows_per_tile,), jnp.float32),
            [pltpu.SemaphoreType.DMA for _ in range(2)],
        ],
        compiler_params=pltpu.CompilerParams(use_tc_tiling_on_sc=False),
    )
    def deg_kernel(eidx_h, ew_h, out_h, accs, idx, ewb, ones_v, zbuf, sem):
        c = lax.axis_index("c")
        s = lax.axis_index("s")
        wid = s * NC + c
        start = wid * blocks_per_w

        z = jnp.zeros((L,), jnp.float32)
        one = jnp.ones((L,), jnp.float32)

        def zb(r, carry):
            zbuf[pl.ds(r * L, L)] = z
            return carry

        lax.fori_loop(0, rows_per_tile // L, zb, 0)
        for i in range(CB // L):
            ones_v[pl.ds(i * L, L)] = one
        r0 = s * rows_per_tile
        for acc in accs:
            pltpu.sync_copy(zbuf, acc.at[pl.ds(r0, rows_per_tile)])

        plsc.subcore_barrier()

        def issue(blk, b):
            pltpu.async_copy(eidx_h.at[blk], idx[b], sem[b])
            pltpu.async_copy(ew_h.at[blk], ewb[b], sem[b])

        def process(b):
            pltpu.make_async_copy(eidx_h.at[0], idx[b], sem[b]).wait()
            pltpu.make_async_copy(ew_h.at[0], ewb[b], sem[b]).wait()
            pltpu.sync_copy(ewb[b], accs[0].at[idx[b].at[0]], add=True)
            pltpu.sync_copy(ones_v, accs[1].at[idx[b].at[0]], add=True)
            pltpu.sync_copy(ewb[b], accs[2].at[idx[b].at[1]], add=True)
            pltpu.sync_copy(ones_v, accs[3].at[idx[b].at[1]], add=True)

        issue(start, 0)
        issue(start + 1, 1)
        npairs = blocks_per_w // 2

        def pair(m, carry):
            process(0)

            @pl.when(m < npairs - 1)
            def _():
                issue(start + 2 * m + 2, 0)

            process(1)

            @pl.when(m < npairs - 1)
            def _():
                issue(start + 2 * m + 3, 1)
            return carry

        lax.fori_loop(0, npairs, pair, 0)
        plsc.subcore_barrier()

        for k, acc in enumerate(accs):
            pltpu.sync_copy(acc.at[pl.ds(r0, rows_per_tile)], zbuf)
            pltpu.sync_copy(
                zbuf, out_h.at[pl.ds((c * 4 + k) * np_ + r0, rows_per_tile)])

    return deg_kernel


def _make_mp_kernel(n, d, n_blk, weighted):
    np_ = _pad_n(n)
    rows_per_tile = np_ // NS
    base_cnt, rem = divmod(n_blk, NS)
    assert rem % 2 == 0
    nx = 2
    dh = d // 2

    scratch = [
        pltpu.VMEM_SHARED((np_, dh), jnp.float32),
        [pltpu.VMEM((2, C2), jnp.int32) for _ in range(2)],
        [pltpu.VMEM((C2, d), jnp.bfloat16) for _ in range(2)],
        pltpu.VMEM((C2, dh), jnp.float32),
        [pltpu.SemaphoreType.DMA for _ in range(2)],
    ]
    if weighted:
        scratch.append([pltpu.VMEM((C2 + L,), jnp.float32) for _ in range(2)])

    @functools.partial(
        pl.kernel,
        mesh=_mesh(),
        out_type=jax.ShapeDtypeStruct((NC, np_, dh), jnp.float32),
        scratch_types=scratch,
        compiler_params=pltpu.CompilerParams(use_tc_tiling_on_sc=False,
                                             needs_layout_passes=False),
    )
    def mp_kernel(feat_h, eidx_h, *rest):
        if weighted:
            ew_h, out_h, agg, idx, rows_g, rows_s, sem, ew_v = rest
        else:
            out_h, agg, idx, rows_g, rows_s, sem = rest
        c = lax.axis_index("c")
        s = lax.axis_index("s")
        half = rem // nx
        cnt = jnp.where(s < half, base_cnt + nx, base_cnt)
        start = s * base_cnt + nx * jnp.minimum(s, half)

        z = jnp.zeros((L,), jnp.float32)

        def zrow(r, carry):
            for jj in range(dh // L):
                rows_s[r, pl.ds(jj * L, L)] = z
            return carry

        lax.fori_loop(0, C2, zrow, 0)
        r0 = s * rows_per_tile
        for q in range(rows_per_tile // C2):
            pltpu.sync_copy(rows_s, agg.at[pl.ds(r0 + q * C2, C2)])
        plsc.subcore_barrier()

        def issue(blk, b):
            pltpu.sync_copy(eidx_h.at[blk], idx[b])
            if weighted:
                pltpu.sync_copy(ew_h.at[blk], ew_v[b].at[pl.ds(0, C2)])
            pltpu.async_copy(feat_h.at[idx[b].at[0]], rows_g[b], sem[b])

        def process(b):
            def widen(ei, cc):
                if weighted:
                    w = ew_v[b][pl.ds(ei, L)][0]
                for gg in range(dh // 32):
                    gsrc = 32 * ((dh // 32) * c + gg)
                    vb = rows_g[b][ei, pl.ds(gsrc, 32)]
                    lo, hi = plsc.unpack(
                        vb, format=plsc.PackFormat.INTERLEAVED,
                        preferred_element_type=jnp.float32)
                    if weighted:
                        lo = lo * w
                        hi = hi * w
                    rows_s[ei, pl.ds(32 * gg, L)] = lo
                    rows_s[ei, pl.ds(32 * gg + L, L)] = hi
                return cc

            lax.fori_loop(0, C2, widen, 0)
            pltpu.sync_copy(rows_s, agg.at[idx[b].at[1]], add=True)

        issue(start, 0)
        issue(start + 1, 1)
        npairs = cnt // 2

        def drain(b):
            pltpu.make_async_copy(feat_h.at[idx[b].at[0]], rows_g[b],
                                  sem[b]).wait()

        def pair(m, carry):
            drain(0)
            process(0)

            @pl.when(m < npairs - 1)
            def _():
                issue(start + 2 * m + 2, 0)

            drain(1)
            process(1)

            @pl.when(m < npairs - 1)
            def _():
                issue(start + 2 * m + 3, 1)
            return carry

        lax.fori_loop(0, npairs, pair, 0)
        plsc.subcore_barrier()

        for q in range(rows_per_tile // C2):
            pltpu.sync_copy(agg.at[pl.ds(r0 + q * C2, C2)], rows_s)
            pltpu.sync_copy(rows_s, out_h.at[c, pl.ds(r0 + q * C2, C2)])

    return mp_kernel


def _tc_norms_mm(deg_ref, x_ref, w1_ref, featn_ref, norms_ref):
    dsum = jnp.sum(deg_ref[...], axis=2)
    norms = jnp.where(dsum > 0, lax.rsqrt(jnp.maximum(dsum, 1e-12)), 0.0)
    norms_ref[...] = norms
    xw = jnp.dot(x_ref[...], w1_ref[...], preferred_element_type=jnp.float32)
    featn_ref[...] = (xw * norms[:, 0:1]).astype(jnp.bfloat16)


def _tc_mid(agg_ref, norms_ref, b1_ref, out_ref):
    norms = norms_ref[...]
    p = agg_ref[...]
    full = jnp.concatenate([p[0], p[1]], axis=1)
    h = full * norms[:, 2:3] + b1_ref[...][None, :]
    h = jnp.maximum(h, 0.0)
    out_ref[...] = (h * norms[:, 1:2]).astype(jnp.bfloat16)


def _tc_final(agg_ref, norms_ref, b2_ref, w2_ref, out_ref):
    p = agg_ref[...]
    full = jnp.concatenate([p[0], p[1]], axis=1)
    f2 = jnp.dot(full, w2_ref[...], preferred_element_type=jnp.float32)
    z = f2 * norms_ref[...][:, 3:4] + b2_ref[...][None, :]
    m = jnp.max(z, axis=1, keepdims=True)
    ez = jnp.exp(z - m)
    out_ref[...] = ez / jnp.sum(ez, axis=1, keepdims=True)


def kernel(x, edge_index, edge_weight, W1, b1, W2, b2):
    n, d_in = x.shape
    e = edge_index.shape[1]
    d_hid = W1.shape[1]
    d_out = W2.shape[1]

    src = edge_index[0].astype(jnp.int32)
    dst = edge_index[1].astype(jnp.int32)

    np_ = _pad_n(n)
    n_blk = e // CB
    eidx = jnp.stack([src.reshape(n_blk, CB), dst.reshape(n_blk, CB)], axis=1)
    ew2d = edge_weight.reshape(n_blk, CB)

    n_blkp = ((n_blk + 2 * NW - 1) // (2 * NW)) * (2 * NW)
    e_pad = (n_blkp - n_blk) * CB
    pad_dst = (n + (jnp.arange(e_pad, dtype=jnp.int32) % (np_ - n))
               ).reshape(-1, CB)
    pad_idx = jnp.stack([jnp.zeros_like(pad_dst), pad_dst], axis=1)
    eidx_p = jnp.concatenate([eidx, pad_idx])
    ew2d_p = jnp.concatenate([ew2d, jnp.zeros((n_blkp - n_blk, CB),
                                              jnp.float32)])

    deg_part = _make_deg_kernel(n, n_blkp)(eidx_p, ew2d_p)
    deg = deg_part.reshape(NC, 4, np_)[:, :, :n].transpose(2, 1, 0)

    featn, norms = pl.pallas_call(
        _tc_norms_mm,
        out_shape=(
            jax.ShapeDtypeStruct((n, d_hid), jnp.bfloat16),
            jax.ShapeDtypeStruct((n, 4), jnp.float32),
        ),
    )(deg, x, W1)

    n_blk2 = e // C2
    eidx2 = jnp.stack([src.reshape(n_blk2, C2), dst.reshape(n_blk2, C2)],
                      axis=1)
    ew2d2 = edge_weight.reshape(n_blk2, C2)
    q = _qperm(d_hid)

    agg1 = _make_mp_kernel(n, d_hid, n_blk2, True)(featn, eidx2,
                                                   ew2d2)[:, :n]

    hn = pl.pallas_call(
        _tc_mid,
        out_shape=jax.ShapeDtypeStruct((n, d_hid), jnp.bfloat16),
    )(agg1, norms, b1[q])

    agg2 = _make_mp_kernel(n, d_hid, n_blk2, False)(hn, eidx2)[:, :n]

    return pl.pallas_call(
        _tc_final,
        out_shape=jax.ShapeDtypeStruct((n, d_out), jnp.float32),
    )(agg2, norms, b2, W2[q[q]])


# speedup vs baseline: 1.3526x; 1.3526x over previous
import functools

import jax
import jax.numpy as jnp
from jax import lax
from jax.experimental import pallas as pl
from jax.experimental.pallas import tpu as pltpu
from jax.experimental.pallas import tpu_sc as plsc

NC = 2
NS = 16
NW = NC * NS
L = 16
C = 80
CB = 128


def _zero2d(ref, nrows, ncols):
    z = jnp.zeros((L,), jnp.float32)

    def body(r, carry):
        for jj in range(ncols // L):
            ref[r, pl.ds(jj * L, L)] = z
        return carry

    lax.fori_loop(0, nrows, body, 0)


def _mesh(nc=NC):
    return plsc.VectorSubcoreMesh(core_axis_name="c", subcore_axis_name="s",
                                  num_cores=nc)


def _pad_n(n):
    return ((n + NS * L - 1) // (NS * L)) * (NS * L)


def _make_deg_kernel(n, n_blk):
    np_ = _pad_n(n)
    rows_per_tile = np_ // NS
    blocks_per_w = n_blk // NW
    assert blocks_per_w % 2 == 0

    @functools.partial(
        pl.kernel,
        mesh=_mesh(),
        out_type=jax.ShapeDtypeStruct((NC * 4 * np_,), jnp.float32),
        scratch_types=[
            [pltpu.VMEM_SHARED((np_,), jnp.float32) for _ in range(4)],
            [pltpu.VMEM((2, CB), jnp.int32) for _ in range(2)],
            [pltpu.VMEM((CB,), jnp.float32) for _ in range(2)],
            pltpu.VMEM((CB,), jnp.float32),
            pltpu.VMEM((rows_per_tile,), jnp.float32),
            [pltpu.SemaphoreType.DMA for _ in range(2)],
        ],
        compiler_params=pltpu.CompilerParams(use_tc_tiling_on_sc=False),
    )
    def deg_kernel(eidx_h, ew_h, out_h, accs, idx, ewb, ones_v, zbuf, sem):
        c = lax.axis_index("c")
        s = lax.axis_index("s")
        wid = s * NC + c
        start = wid * blocks_per_w

        z = jnp.zeros((L,), jnp.float32)
        one = jnp.ones((L,), jnp.float32)

        def zb(r, carry):
            zbuf[pl.ds(r * L, L)] = z
            return carry

        lax.fori_loop(0, rows_per_tile // L, zb, 0)
        for i in range(CB // L):
            ones_v[pl.ds(i * L, L)] = one
        r0 = s * rows_per_tile
        for acc in accs:
            pltpu.sync_copy(zbuf, acc.at[pl.ds(r0, rows_per_tile)])

        plsc.subcore_barrier()

        def issue(blk, b):
            pltpu.async_copy(eidx_h.at[blk], idx[b], sem[b])
            pltpu.async_copy(ew_h.at[blk], ewb[b], sem[b])

        def process(b):
            pltpu.make_async_copy(eidx_h.at[0], idx[b], sem[b]).wait()
            pltpu.make_async_copy(ew_h.at[0], ewb[b], sem[b]).wait()
            pltpu.sync_copy(ewb[b], accs[0].at[idx[b].at[0]], add=True)
            pltpu.sync_copy(ones_v, accs[1].at[idx[b].at[0]], add=True)
            pltpu.sync_copy(ewb[b], accs[2].at[idx[b].at[1]], add=True)
            pltpu.sync_copy(ones_v, accs[3].at[idx[b].at[1]], add=True)

        issue(start, 0)
        issue(start + 1, 1)
        npairs = blocks_per_w // 2

        def pair(m, carry):
            process(0)

            @pl.when(m < npairs - 1)
            def _():
                issue(start + 2 * m + 2, 0)

            process(1)

            @pl.when(m < npairs - 1)
            def _():
                issue(start + 2 * m + 3, 1)
            return carry

        lax.fori_loop(0, npairs, pair, 0)
        plsc.subcore_barrier()

        for k, acc in enumerate(accs):
            pltpu.sync_copy(acc.at[pl.ds(r0, rows_per_tile)], zbuf)
            pltpu.sync_copy(
                zbuf, out_h.at[pl.ds((c * 4 + k) * np_ + r0, rows_per_tile)])

    return deg_kernel


def _make_mp_kernel(n, d, n_blk, weighted):
    np_ = _pad_n(n)
    rows_per_tile = np_ // NS
    base_cnt, rem = divmod(n_blk, NS)
    assert rem % 2 == 0
    nx = 2

    scratch = [
        pltpu.VMEM_SHARED((np_, d), jnp.float32),
        [pltpu.VMEM((2, CB), jnp.int32) for _ in range(2)],
        [pltpu.VMEM((CB, d), jnp.float32) for _ in range(2)],
        [pltpu.SemaphoreType.DMA for _ in range(2)],
    ]
    if weighted:
        scratch.append([pltpu.VMEM((CB + L,), jnp.float32) for _ in range(2)])

    @functools.partial(
        pl.kernel,
        mesh=_mesh(1),
        out_type=jax.ShapeDtypeStruct((np_, d), jnp.float32),
        scratch_types=scratch,
        compiler_params=pltpu.CompilerParams(use_tc_tiling_on_sc=False),
    )
    def mp_kernel(feat_h, eidx_h, *rest):
        if weighted:
            ew_h, out_h, agg, idx, rows, sem, ew_v = rest
        else:
            out_h, agg, idx, rows, sem = rest
        s = lax.axis_index("s")
        half = rem // nx
        cnt = jnp.where(s < half, base_cnt + nx, base_cnt)
        start = s * base_cnt + nx * jnp.minimum(s, half)

        _zero2d(rows[0], CB, d)
        r0 = s * rows_per_tile
        for q in range(rows_per_tile // CB):
            pltpu.sync_copy(rows[0], agg.at[pl.ds(r0 + q * CB, CB)])
        plsc.subcore_barrier()

        def issue(blk, b):
            pltpu.sync_copy(eidx_h.at[blk], idx[b])
            if weighted:
                pltpu.sync_copy(ew_h.at[blk], ew_v[b].at[pl.ds(0, CB)])
            return pltpu.async_copy(feat_h.at[idx[b].at[0]], rows[b], sem[b])

        def scale(b):
            if weighted:
                def body(ei, cc):
                    w = ew_v[b][pl.ds(ei, L)][0]
                    for jj in range(d // L):
                        sl = pl.ds(jj * L, L)
                        rows[b][ei, sl] = rows[b][ei, sl] * w
                    return cc
                lax.fori_loop(0, CB, body, 0)

        def scatter(b):
            pltpu.sync_copy(rows[b], agg.at[idx[b].at[1]], add=True)

        issue(start, 0)
        issue(start + 1, 1)

        npairs = cnt // 2

        def drain(b):
            pltpu.make_async_copy(feat_h.at[idx[b].at[0]], rows[b],
                                  sem[b]).wait()

        def pair(m, carry):
            drain(0)
            scale(0)
            drain(1)
            scale(1)
            scatter(0)

            @pl.when(m < npairs - 1)
            def _():
                issue(start + 2 * m + 2, 0)

            scatter(1)

            @pl.when(m < npairs - 1)
            def _():
                issue(start + 2 * m + 3, 1)
            return carry

        lax.fori_loop(0, npairs, pair, 0)
        plsc.subcore_barrier()

        for q in range(rows_per_tile // CB):
            pltpu.sync_copy(agg.at[pl.ds(r0 + q * CB, CB)], rows[0])
            pltpu.sync_copy(rows[0], out_h.at[pl.ds(r0 + q * CB, CB)])

    return mp_kernel


def _tc_norms_mm(deg_ref, x_ref, w1_ref, featn_ref, norms_ref):
    dsum = jnp.sum(deg_ref[...], axis=2)
    norms = jnp.where(dsum > 0, lax.rsqrt(jnp.maximum(dsum, 1e-12)), 0.0)
    norms_ref[...] = norms
    xw = jnp.dot(x_ref[...], w1_ref[...], preferred_element_type=jnp.float32)
    featn_ref[...] = xw * norms[:, 0:1]


def _tc_mid(agg_ref, norms_ref, b1_ref, out_ref):
    norms = norms_ref[...]
    h = agg_ref[...] * norms[:, 2:3] + b1_ref[...][None, :]
    h = jnp.maximum(h, 0.0)
    out_ref[...] = h * norms[:, 1:2]


def _tc_final(agg_ref, norms_ref, b2_ref, w2_ref, out_ref):
    f2 = jnp.dot(agg_ref[...], w2_ref[...], preferred_element_type=jnp.float32)
    z = f2 * norms_ref[...][:, 3:4] + b2_ref[...][None, :]
    m = jnp.max(z, axis=1, keepdims=True)
    ez = jnp.exp(z - m)
    out_ref[...] = ez / jnp.sum(ez, axis=1, keepdims=True)


def kernel(x, edge_index, edge_weight, W1, b1, W2, b2):
    n, d_in = x.shape
    e = edge_index.shape[1]
    d_hid = W1.shape[1]
    d_out = W2.shape[1]

    src = edge_index[0].astype(jnp.int32)
    dst = edge_index[1].astype(jnp.int32)

    np_ = _pad_n(n)
    n_blk = e // CB
    eidx = jnp.stack([src.reshape(n_blk, CB), dst.reshape(n_blk, CB)], axis=1)
    ew2d = edge_weight.reshape(n_blk, CB)

    n_blkp = ((n_blk + 2 * NW - 1) // (2 * NW)) * (2 * NW)
    e_pad = (n_blkp - n_blk) * CB
    pad_dst = (n + (jnp.arange(e_pad, dtype=jnp.int32) % (np_ - n))
               ).reshape(-1, CB)
    pad_idx = jnp.stack([jnp.zeros_like(pad_dst), pad_dst], axis=1)
    eidx_p = jnp.concatenate([eidx, pad_idx])
    ew2d_p = jnp.concatenate([ew2d, jnp.zeros((n_blkp - n_blk, CB),
                                              jnp.float32)])

    deg_part = _make_deg_kernel(n, n_blkp)(eidx_p, ew2d_p)
    deg = deg_part.reshape(NC, 4, np_)[:, :, :n].transpose(2, 1, 0)

    featn, norms = pl.pallas_call(
        _tc_norms_mm,
        out_shape=(
            jax.ShapeDtypeStruct((n, d_hid), jnp.float32),
            jax.ShapeDtypeStruct((n, 4), jnp.float32),
        ),
    )(deg, x, W1)

    agg1 = _make_mp_kernel(n, d_hid, n_blk, True)(featn, eidx, ew2d)[:n]

    hn = pl.pallas_call(
        _tc_mid,
        out_shape=jax.ShapeDtypeStruct((n, d_hid), jnp.float32),
    )(agg1, norms, b1)

    agg2 = _make_mp_kernel(n, d_hid, n_blk, False)(hn, eidx)[:n]

    return pl.pallas_call(
        _tc_final,
        out_shape=jax.ShapeDtypeStruct((n, d_out), jnp.float32),
    )(agg2, norms, b2, W2)


# speedup vs baseline: 1.4969x; 1.1067x over previous
import functools

import jax
import jax.numpy as jnp
from jax import lax
from jax.experimental import pallas as pl
from jax.experimental.pallas import tpu as pltpu
from jax.experimental.pallas import tpu_sc as plsc

NC = 2
NS = 16
NW = NC * NS
L = 16
C = 80
CB = 128


def _zero2d(ref, nrows, ncols):
    z = jnp.zeros((L,), jnp.float32)

    def body(r, carry):
        for jj in range(ncols // L):
            ref[r, pl.ds(jj * L, L)] = z
        return carry

    lax.fori_loop(0, nrows, body, 0)


def _mesh(nc=NC):
    return plsc.VectorSubcoreMesh(core_axis_name="c", subcore_axis_name="s",
                                  num_cores=nc)


def _pad_n(n):
    return ((n + NS * L - 1) // (NS * L)) * (NS * L)


def _make_deg_kernel(n, n_blk):
    np_ = _pad_n(n)
    rows_per_tile = np_ // NS
    blocks_per_w = n_blk // NW
    assert blocks_per_w % 2 == 0

    @functools.partial(
        pl.kernel,
        mesh=_mesh(),
        out_type=jax.ShapeDtypeStruct((NC * 4 * np_,), jnp.float32),
        scratch_types=[
            [pltpu.VMEM_SHARED((np_,), jnp.float32) for _ in range(4)],
            [pltpu.VMEM((2, CB), jnp.int32) for _ in range(2)],
            [pltpu.VMEM((CB,), jnp.float32) for _ in range(2)],
            pltpu.VMEM((CB,), jnp.float32),
            pltpu.VMEM((rows_per_tile,), jnp.float32),
            [pltpu.SemaphoreType.DMA for _ in range(2)],
        ],
        compiler_params=pltpu.CompilerParams(use_tc_tiling_on_sc=False),
    )
    def deg_kernel(eidx_h, ew_h, out_h, accs, idx, ewb, ones_v, zbuf, sem):
        c = lax.axis_index("c")
        s = lax.axis_index("s")
        wid = s * NC + c
        start = wid * blocks_per_w

        z = jnp.zeros((L,), jnp.float32)
        one = jnp.ones((L,), jnp.float32)

        def zb(r, carry):
            zbuf[pl.ds(r * L, L)] = z
            return carry

        lax.fori_loop(0, rows_per_tile // L, zb, 0)
        for i in range(CB // L):
            ones_v[pl.ds(i * L, L)] = one
        r0 = s * rows_per_tile
        for acc in accs:
            pltpu.sync_copy(zbuf, acc.at[pl.ds(r0, rows_per_tile)])

        plsc.subcore_barrier()

        def issue(blk, b):
            pltpu.async_copy(eidx_h.at[blk], idx[b], sem[b])
            pltpu.async_copy(ew_h.at[blk], ewb[b], sem[b])

        def process(b):
            pltpu.make_async_copy(eidx_h.at[0], idx[b], sem[b]).wait()
            pltpu.make_async_copy(ew_h.at[0], ewb[b], sem[b]).wait()
            pltpu.sync_copy(ewb[b], accs[0].at[idx[b].at[0]], add=True)
            pltpu.sync_copy(ones_v, accs[1].at[idx[b].at[0]], add=True)
            pltpu.sync_copy(ewb[b], accs[2].at[idx[b].at[1]], add=True)
            pltpu.sync_copy(ones_v, accs[3].at[idx[b].at[1]], add=True)

        issue(start, 0)
        issue(start + 1, 1)
        npairs = blocks_per_w // 2

        def pair(m, carry):
            process(0)

            @pl.when(m < npairs - 1)
            def _():
                issue(start + 2 * m + 2, 0)

            process(1)

            @pl.when(m < npairs - 1)
            def _():
                issue(start + 2 * m + 3, 1)
            return carry

        lax.fori_loop(0, npairs, pair, 0)
        plsc.subcore_barrier()

        for k, acc in enumerate(accs):
            pltpu.sync_copy(acc.at[pl.ds(r0, rows_per_tile)], zbuf)
            pltpu.sync_copy(
                zbuf, out_h.at[pl.ds((c * 4 + k) * np_ + r0, rows_per_tile)])

    return deg_kernel


def _make_mp_kernel(n, d, n_blk, weighted):
    np_ = _pad_n(n)
    rows_per_tile = np_ // NS
    base_cnt, rem = divmod(n_blk, NS)
    assert rem % 2 == 0
    nx = 2

    scratch = [
        pltpu.VMEM_SHARED((np_, d), jnp.float32),
        [pltpu.VMEM((2, CB), jnp.int32) for _ in range(2)],
        [pltpu.VMEM((CB, d), jnp.float32) for _ in range(2)],
        [pltpu.SemaphoreType.DMA for _ in range(2)],
    ]
    if weighted:
        scratch.append([pltpu.VMEM((CB + L,), jnp.float32) for _ in range(2)])

    @functools.partial(
        pl.kernel,
        mesh=_mesh(1),
        out_type=jax.ShapeDtypeStruct((np_, d), jnp.float32),
        scratch_types=scratch,
        compiler_params=pltpu.CompilerParams(use_tc_tiling_on_sc=False),
    )
    def mp_kernel(feat_h, eidx_h, *rest):
        if weighted:
            ew_h, out_h, agg, idx, rows, sem, ew_v = rest
        else:
            out_h, agg, idx, rows, sem = rest
        s = lax.axis_index("s")
        half = rem // nx
        cnt = jnp.where(s < half, base_cnt + nx, base_cnt)
        start = s * base_cnt + nx * jnp.minimum(s, half)

        _zero2d(rows[0], CB, d)
        r0 = s * rows_per_tile
        for q in range(rows_per_tile // CB):
            pltpu.sync_copy(rows[0], agg.at[pl.ds(r0 + q * CB, CB)])
        plsc.subcore_barrier()

        def issue(blk, b):
            pltpu.sync_copy(eidx_h.at[blk], idx[b])
            if weighted:
                pltpu.sync_copy(ew_h.at[blk], ew_v[b].at[pl.ds(0, CB)])
            return pltpu.async_copy(feat_h.at[idx[b].at[0]], rows[b], sem[b])

        def process(b):
            if weighted:
                def scale(ei, cc):
                    w = ew_v[b][pl.ds(ei, L)][0]
                    for jj in range(d // L):
                        sl = pl.ds(jj * L, L)
                        rows[b][ei, sl] = rows[b][ei, sl] * w
                    return cc
                lax.fori_loop(0, CB, scale, 0)
            pltpu.sync_copy(rows[b], agg.at[idx[b].at[1]], add=True)

        issue(start, 0)
        issue(start + 1, 1)

        npairs = cnt // 2

        def drain(b):
            pltpu.make_async_copy(feat_h.at[idx[b].at[0]], rows[b],
                                  sem[b]).wait()

        def pair(m, carry):
            drain(0)
            process(0)

            @pl.when(m < npairs - 1)
            def _():
                issue(start + 2 * m + 2, 0)

            drain(1)
            process(1)

            @pl.when(m < npairs - 1)
            def _():
                issue(start + 2 * m + 3, 1)
            return carry

        lax.fori_loop(0, npairs, pair, 0)
        plsc.subcore_barrier()

        for q in range(rows_per_tile // CB):
            pltpu.sync_copy(agg.at[pl.ds(r0 + q * CB, CB)], rows[0])
            pltpu.sync_copy(rows[0], out_h.at[pl.ds(r0 + q * CB, CB)])

    return mp_kernel


def _tc_norms_mm(deg_ref, x_ref, w1_ref, featn_ref, norms_ref):
    dsum = jnp.sum(deg_ref[...], axis=2)
    norms = jnp.where(dsum > 0, lax.rsqrt(jnp.maximum(dsum, 1e-12)), 0.0)
    norms_ref[...] = norms
    xw = jnp.dot(x_ref[...], w1_ref[...], preferred_element_type=jnp.float32)
    featn_ref[...] = xw * norms[:, 0:1]


def _tc_mid(agg_ref, norms_ref, b1_ref, out_ref):
    norms = norms_ref[...]
    h = agg_ref[...] * norms[:, 2:3] + b1_ref[...][None, :]
    h = jnp.maximum(h, 0.0)
    out_ref[...] = h * norms[:, 1:2]


def _tc_final(agg_ref, norms_ref, b2_ref, w2_ref, out_ref):
    f2 = jnp.dot(agg_ref[...], w2_ref[...], preferred_element_type=jnp.float32)
    z = f2 * norms_ref[...][:, 3:4] + b2_ref[...][None, :]
    m = jnp.max(z, axis=1, keepdims=True)
    ez = jnp.exp(z - m)
    out_ref[...] = ez / jnp.sum(ez, axis=1, keepdims=True)


def kernel(x, edge_index, edge_weight, W1, b1, W2, b2):
    n, d_in = x.shape
    e = edge_index.shape[1]
    d_hid = W1.shape[1]
    d_out = W2.shape[1]

    src = edge_index[0].astype(jnp.int32)
    dst = edge_index[1].astype(jnp.int32)

    np_ = _pad_n(n)
    n_blk = e // CB
    eidx = jnp.stack([src.reshape(n_blk, CB), dst.reshape(n_blk, CB)], axis=1)
    ew2d = edge_weight.reshape(n_blk, CB)

    n_blkp = ((n_blk + 2 * NW - 1) // (2 * NW)) * (2 * NW)
    e_pad = (n_blkp - n_blk) * CB
    pad_dst = (n + (jnp.arange(e_pad, dtype=jnp.int32) % (np_ - n))
               ).reshape(-1, CB)
    pad_idx = jnp.stack([jnp.zeros_like(pad_dst), pad_dst], axis=1)
    eidx_p = jnp.concatenate([eidx, pad_idx])
    ew2d_p = jnp.concatenate([ew2d, jnp.zeros((n_blkp - n_blk, CB),
                                              jnp.float32)])

    deg_part = _make_deg_kernel(n, n_blkp)(eidx_p, ew2d_p)
    deg = deg_part.reshape(NC, 4, np_)[:, :, :n].transpose(2, 1, 0)

    featn, norms = pl.pallas_call(
        _tc_norms_mm,
        out_shape=(
            jax.ShapeDtypeStruct((n, d_hid), jnp.float32),
            jax.ShapeDtypeStruct((n, 4), jnp.float32),
        ),
    )(deg, x, W1)

    agg1 = _make_mp_kernel(n, d_hid, n_blk, True)(featn, eidx, ew2d)[:n]

    hn = pl.pallas_call(
        _tc_mid,
        out_shape=jax.ShapeDtypeStruct((n, d_hid), jnp.float32),
    )(agg1, norms, b1)

    agg2 = _make_mp_kernel(n, d_hid, n_blk, False)(hn, eidx)[:n]

    return pl.pallas_call(
        _tc_final,
        out_shape=jax.ShapeDtypeStruct((n, d_out), jnp.float32),
    )(agg2, norms, b2, W2)


# speedup vs baseline: 1.9588x; 1.3085x over previous
import functools

import jax
import jax.numpy as jnp
from jax import lax
from jax.experimental import pallas as pl
from jax.experimental.pallas import tpu as pltpu
from jax.experimental.pallas import tpu_sc as plsc

NC = 2
NS = 16
NW = NC * NS
L = 16
C = 80
CB = 128


def _zero2d(ref, nrows, ncols):
    z = jnp.zeros((L,), jnp.float32)

    def body(r, carry):
        for jj in range(ncols // L):
            ref[r, pl.ds(jj * L, L)] = z
        return carry

    lax.fori_loop(0, nrows, body, 0)


def _mesh(nc=NC):
    return plsc.VectorSubcoreMesh(core_axis_name="c", subcore_axis_name="s",
                                  num_cores=nc)


def _pad_n(n):
    return ((n + NS * L - 1) // (NS * L)) * (NS * L)


def _make_deg_kernel(n, n_blk):
    np_ = _pad_n(n)
    rows_per_tile = np_ // NS
    blocks_per_w = n_blk // NW
    assert blocks_per_w % 2 == 0

    @functools.partial(
        pl.kernel,
        mesh=_mesh(),
        out_type=jax.ShapeDtypeStruct((NC * 4 * np_,), jnp.float32),
        scratch_types=[
            [pltpu.VMEM_SHARED((np_,), jnp.float32) for _ in range(4)],
            [pltpu.VMEM((2, CB), jnp.int32) for _ in range(2)],
            [pltpu.VMEM((CB,), jnp.float32) for _ in range(2)],
            pltpu.VMEM((CB,), jnp.float32),
            pltpu.VMEM((rows_per_tile,), jnp.float32),
            [pltpu.SemaphoreType.DMA for _ in range(2)],
        ],
        compiler_params=pltpu.CompilerParams(use_tc_tiling_on_sc=False),
    )
    def deg_kernel(eidx_h, ew_h, out_h, accs, idx, ewb, ones_v, zbuf, sem):
        c = lax.axis_index("c")
        s = lax.axis_index("s")
        wid = s * NC + c
        start = wid * blocks_per_w

        z = jnp.zeros((L,), jnp.float32)
        one = jnp.ones((L,), jnp.float32)

        def zb(r, carry):
            zbuf[pl.ds(r * L, L)] = z
            return carry

        lax.fori_loop(0, rows_per_tile // L, zb, 0)
        for i in range(CB // L):
            ones_v[pl.ds(i * L, L)] = one
        r0 = s * rows_per_tile
        for acc in accs:
            pltpu.sync_copy(zbuf, acc.at[pl.ds(r0, rows_per_tile)])

        plsc.subcore_barrier()

        def issue(blk, b):
            pltpu.async_copy(eidx_h.at[blk], idx[b], sem[b])
            pltpu.async_copy(ew_h.at[blk], ewb[b], sem[b])

        def process(b):
            pltpu.make_async_copy(eidx_h.at[0], idx[b], sem[b]).wait()
            pltpu.make_async_copy(ew_h.at[0], ewb[b], sem[b]).wait()
            pltpu.sync_copy(ewb[b], accs[0].at[idx[b].at[0]], add=True)
            pltpu.sync_copy(ones_v, accs[1].at[idx[b].at[0]], add=True)
            pltpu.sync_copy(ewb[b], accs[2].at[idx[b].at[1]], add=True)
            pltpu.sync_copy(ones_v, accs[3].at[idx[b].at[1]], add=True)

        issue(start, 0)
        issue(start + 1, 1)
        npairs = blocks_per_w // 2

        def pair(m, carry):
            process(0)

            @pl.when(m < npairs - 1)
            def _():
                issue(start + 2 * m + 2, 0)

            process(1)

            @pl.when(m < npairs - 1)
            def _():
                issue(start + 2 * m + 3, 1)
            return carry

        lax.fori_loop(0, npairs, pair, 0)
        plsc.subcore_barrier()

        for k, acc in enumerate(accs):
            pltpu.sync_copy(acc.at[pl.ds(r0, rows_per_tile)], zbuf)
            pltpu.sync_copy(
                zbuf, out_h.at[pl.ds((c * 4 + k) * np_ + r0, rows_per_tile)])

    return deg_kernel


def _make_mp_kernel(n, d, n_blk, weighted):
    np_ = _pad_n(n)
    rows_per_tile = np_ // NS
    base_cnt, rem = divmod(n_blk, NS)
    assert rem % 2 == 0
    nx = 2

    scratch = [
        pltpu.VMEM_SHARED((np_, d), jnp.float32),
        [pltpu.VMEM((2, CB), jnp.int32) for _ in range(2)],
        [pltpu.VMEM((CB, d), jnp.float32) for _ in range(2)],
        [pltpu.SemaphoreType.DMA for _ in range(2)],
    ]
    if weighted:
        scratch.append([pltpu.VMEM((CB + L,), jnp.float32) for _ in range(2)])

    @functools.partial(
        pl.kernel,
        mesh=_mesh(1),
        out_type=jax.ShapeDtypeStruct((np_, d), jnp.float32),
        scratch_types=scratch,
        compiler_params=pltpu.CompilerParams(use_tc_tiling_on_sc=False),
    )
    def mp_kernel(feat_h, eidx_h, *rest):
        if weighted:
            ew_h, out_h, agg, idx, rows, sem, ew_v = rest
        else:
            out_h, agg, idx, rows, sem = rest
        s = lax.axis_index("s")
        half = rem // nx
        cnt = jnp.where(s < half, base_cnt + nx, base_cnt)
        start = s * base_cnt + nx * jnp.minimum(s, half)

        _zero2d(rows[0], CB, d)
        r0 = s * rows_per_tile
        for q in range(rows_per_tile // CB):
            pltpu.sync_copy(rows[0], agg.at[pl.ds(r0 + q * CB, CB)])
        plsc.subcore_barrier()

        def issue(blk, b):
            pltpu.sync_copy(eidx_h.at[blk], idx[b])
            if weighted:
                pltpu.sync_copy(ew_h.at[blk], ew_v[b].at[pl.ds(0, CB)])
            return pltpu.async_copy(feat_h.at[idx[b].at[0]], rows[b], sem[b])

        def process(b):
            if weighted:
                @functools.partial(plsc.parallel_loop, 0, CB, unroll=2)
                def _(ei):
                    w = ew_v[b][pl.ds(ei, L)][0]
                    for jj in range(d // L):
                        sl = pl.ds(jj * L, L)
                        rows[b][ei, sl] = rows[b][ei, sl] * w
            pltpu.sync_copy(rows[b], agg.at[idx[b].at[1]], add=True)

        issue(start, 0)
        issue(start + 1, 1)

        npairs = cnt // 2

        def drain(b):
            pltpu.make_async_copy(feat_h.at[idx[b].at[0]], rows[b],
                                  sem[b]).wait()

        def pair(m, carry):
            drain(0)
            process(0)

            @pl.when(m < npairs - 1)
            def _():
                issue(start + 2 * m + 2, 0)

            drain(1)
            process(1)

            @pl.when(m < npairs - 1)
            def _():
                issue(start + 2 * m + 3, 1)
            return carry

        lax.fori_loop(0, npairs, pair, 0)
        plsc.subcore_barrier()

        for q in range(rows_per_tile // CB):
            pltpu.sync_copy(agg.at[pl.ds(r0 + q * CB, CB)], rows[0])
            pltpu.sync_copy(rows[0], out_h.at[pl.ds(r0 + q * CB, CB)])

    return mp_kernel


def _tc_norms_mm(deg_ref, x_ref, w1_ref, featn_ref, norms_ref):
    dsum = jnp.sum(deg_ref[...], axis=2)
    norms = jnp.where(dsum > 0, lax.rsqrt(jnp.maximum(dsum, 1e-12)), 0.0)
    norms_ref[...] = norms
    xw = jnp.dot(x_ref[...], w1_ref[...], preferred_element_type=jnp.float32)
    featn_ref[...] = xw * norms[:, 0:1]


def _tc_mid(agg_ref, norms_ref, b1_ref, out_ref):
    norms = norms_ref[...]
    h = agg_ref[...] * norms[:, 2:3] + b1_ref[...][None, :]
    h = jnp.maximum(h, 0.0)
    out_ref[...] = h * norms[:, 1:2]


def _tc_final(agg_ref, norms_ref, b2_ref, w2_ref, out_ref):
    f2 = jnp.dot(agg_ref[...], w2_ref[...], preferred_element_type=jnp.float32)
    z = f2 * norms_ref[...][:, 3:4] + b2_ref[...][None, :]
    m = jnp.max(z, axis=1, keepdims=True)
    ez = jnp.exp(z - m)
    out_ref[...] = ez / jnp.sum(ez, axis=1, keepdims=True)


def kernel(x, edge_index, edge_weight, W1, b1, W2, b2):
    n, d_in = x.shape
    e = edge_index.shape[1]
    d_hid = W1.shape[1]
    d_out = W2.shape[1]

    src = edge_index[0].astype(jnp.int32)
    dst = edge_index[1].astype(jnp.int32)

    np_ = _pad_n(n)
    n_blk = e // CB
    eidx = jnp.stack([src.reshape(n_blk, CB), dst.reshape(n_blk, CB)], axis=1)
    ew2d = edge_weight.reshape(n_blk, CB)

    n_blkp = ((n_blk + 2 * NW - 1) // (2 * NW)) * (2 * NW)
    e_pad = (n_blkp - n_blk) * CB
    pad_dst = (n + (jnp.arange(e_pad, dtype=jnp.int32) % (np_ - n))
               ).reshape(-1, CB)
    pad_idx = jnp.stack([jnp.zeros_like(pad_dst), pad_dst], axis=1)
    eidx_p = jnp.concatenate([eidx, pad_idx])
    ew2d_p = jnp.concatenate([ew2d, jnp.zeros((n_blkp - n_blk, CB),
                                              jnp.float32)])

    deg_part = _make_deg_kernel(n, n_blkp)(eidx_p, ew2d_p)
    deg = deg_part.reshape(NC, 4, np_)[:, :, :n].transpose(2, 1, 0)

    featn, norms = pl.pallas_call(
        _tc_norms_mm,
        out_shape=(
            jax.ShapeDtypeStruct((n, d_hid), jnp.float32),
            jax.ShapeDtypeStruct((n, 4), jnp.float32),
        ),
    )(deg, x, W1)

    agg1 = _make_mp_kernel(n, d_hid, n_blk, True)(featn, eidx, ew2d)[:n]

    hn = pl.pallas_call(
        _tc_mid,
        out_shape=jax.ShapeDtypeStruct((n, d_hid), jnp.float32),
    )(agg1, norms, b1)

    agg2 = _make_mp_kernel(n, d_hid, n_blk, False)(hn, eidx)[:n]

    return pl.pallas_call(
        _tc_final,
        out_shape=jax.ShapeDtypeStruct((n, d_out), jnp.float32),
    )(agg2, norms, b2, W2)


# speedup vs baseline: 1.9602x; 1.0008x over previous
import functools

import jax
import jax.numpy as jnp
from jax import lax
from jax.experimental import pallas as pl
from jax.experimental.pallas import tpu as pltpu
from jax.experimental.pallas import tpu_sc as plsc

NC = 2
NS = 16
NW = NC * NS
L = 16
C = 80
CB = 128


def _zero2d(ref, nrows, ncols):
    z = jnp.zeros((L,), jnp.float32)

    def body(r, carry):
        for jj in range(ncols // L):
            ref[r, pl.ds(jj * L, L)] = z
        return carry

    lax.fori_loop(0, nrows, body, 0)


def _mesh(nc=NC):
    return plsc.VectorSubcoreMesh(core_axis_name="c", subcore_axis_name="s",
                                  num_cores=nc)


def _pad_n(n):
    return ((n + NS * L - 1) // (NS * L)) * (NS * L)


def _make_deg_kernel(n, n_blk):
    np_ = _pad_n(n)
    rows_per_tile = np_ // NS
    blocks_per_w = n_blk // NW
    assert blocks_per_w % 2 == 0

    @functools.partial(
        pl.kernel,
        mesh=_mesh(),
        out_type=jax.ShapeDtypeStruct((NC * 4 * np_,), jnp.float32),
        scratch_types=[
            [pltpu.VMEM_SHARED((np_,), jnp.float32) for _ in range(4)],
            [pltpu.VMEM((2, CB), jnp.int32) for _ in range(2)],
            [pltpu.VMEM((CB,), jnp.float32) for _ in range(2)],
            pltpu.VMEM((CB,), jnp.float32),
            pltpu.VMEM((rows_per_tile,), jnp.float32),
            [pltpu.SemaphoreType.DMA for _ in range(2)],
        ],
        compiler_params=pltpu.CompilerParams(use_tc_tiling_on_sc=False),
    )
    def deg_kernel(eidx_h, ew_h, out_h, accs, idx, ewb, ones_v, zbuf, sem):
        c = lax.axis_index("c")
        s = lax.axis_index("s")
        wid = s * NC + c
        start = wid * blocks_per_w

        z = jnp.zeros((L,), jnp.float32)
        one = jnp.ones((L,), jnp.float32)

        def zb(r, carry):
            zbuf[pl.ds(r * L, L)] = z
            return carry

        lax.fori_loop(0, rows_per_tile // L, zb, 0)
        for i in range(CB // L):
            ones_v[pl.ds(i * L, L)] = one
        r0 = s * rows_per_tile
        for acc in accs:
            pltpu.sync_copy(zbuf, acc.at[pl.ds(r0, rows_per_tile)])

        plsc.subcore_barrier()

        def issue(blk, b):
            pltpu.async_copy(eidx_h.at[blk], idx[b], sem[b])
            pltpu.async_copy(ew_h.at[blk], ewb[b], sem[b])

        def process(b):
            pltpu.make_async_copy(eidx_h.at[0], idx[b], sem[b]).wait()
            pltpu.make_async_copy(ew_h.at[0], ewb[b], sem[b]).wait()
            pltpu.sync_copy(ewb[b], accs[0].at[idx[b].at[0]], add=True)
            pltpu.sync_copy(ones_v, accs[1].at[idx[b].at[0]], add=True)
            pltpu.sync_copy(ewb[b], accs[2].at[idx[b].at[1]], add=True)
            pltpu.sync_copy(ones_v, accs[3].at[idx[b].at[1]], add=True)

        issue(start, 0)
        issue(start + 1, 1)
        npairs = blocks_per_w // 2

        def pair(m, carry):
            process(0)

            @pl.when(m < npairs - 1)
            def _():
                issue(start + 2 * m + 2, 0)

            process(1)

            @pl.when(m < npairs - 1)
            def _():
                issue(start + 2 * m + 3, 1)
            return carry

        lax.fori_loop(0, npairs, pair, 0)
        plsc.subcore_barrier()

        for k, acc in enumerate(accs):
            pltpu.sync_copy(acc.at[pl.ds(r0, rows_per_tile)], zbuf)
            pltpu.sync_copy(
                zbuf, out_h.at[pl.ds((c * 4 + k) * np_ + r0, rows_per_tile)])

    return deg_kernel


def _make_mp_kernel(n, d, n_blk, weighted):
    np_ = _pad_n(n)
    rows_per_tile = np_ // NS
    base_cnt, rem = divmod(n_blk, NS)
    assert rem % 2 == 0
    nx = 2

    scratch = [
        pltpu.VMEM_SHARED((np_, d), jnp.float32),
        [pltpu.VMEM((2, CB), jnp.int32) for _ in range(2)],
        [pltpu.VMEM((CB, d), jnp.float32) for _ in range(2)],
        [pltpu.SemaphoreType.DMA for _ in range(2)],
    ]
    if weighted:
        scratch.append([pltpu.VMEM((CB + L,), jnp.float32) for _ in range(2)])

    @functools.partial(
        pl.kernel,
        mesh=_mesh(1),
        out_type=jax.ShapeDtypeStruct((np_, d), jnp.float32),
        scratch_types=scratch,
        compiler_params=pltpu.CompilerParams(use_tc_tiling_on_sc=False),
    )
    def mp_kernel(feat_h, eidx_h, *rest):
        if weighted:
            ew_h, out_h, agg, idx, rows, sem, ew_v = rest
        else:
            out_h, agg, idx, rows, sem = rest
        s = lax.axis_index("s")
        half = rem // nx
        cnt = jnp.where(s < half, base_cnt + nx, base_cnt)
        start = s * base_cnt + nx * jnp.minimum(s, half)

        _zero2d(rows[0], CB, d)
        r0 = s * rows_per_tile
        for q in range(rows_per_tile // CB):
            pltpu.sync_copy(rows[0], agg.at[pl.ds(r0 + q * CB, CB)])
        plsc.subcore_barrier()

        def issue(blk, b):
            pltpu.sync_copy(eidx_h.at[blk], idx[b])
            if weighted:
                pltpu.sync_copy(ew_h.at[blk], ew_v[b].at[pl.ds(0, CB)])
            return pltpu.async_copy(feat_h.at[idx[b].at[0]], rows[b], sem[b])

        def process(b):
            if weighted:
                @functools.partial(plsc.parallel_loop, 0, CB)
                def _(ei):
                    w = ew_v[b][pl.ds(ei, L)][0]
                    for jj in range(d // L):
                        sl = pl.ds(jj * L, L)
                        rows[b][ei, sl] = rows[b][ei, sl] * w
            pltpu.sync_copy(rows[b], agg.at[idx[b].at[1]], add=True)

        issue(start, 0)
        issue(start + 1, 1)

        npairs = cnt // 2

        def drain(b):
            pltpu.make_async_copy(feat_h.at[idx[b].at[0]], rows[b],
                                  sem[b]).wait()

        def pair(m, carry):
            drain(0)
            process(0)

            @pl.when(m < npairs - 1)
            def _():
                issue(start + 2 * m + 2, 0)

            drain(1)
            process(1)

            @pl.when(m < npairs - 1)
            def _():
                issue(start + 2 * m + 3, 1)
            return carry

        lax.fori_loop(0, npairs, pair, 0)
        plsc.subcore_barrier()

        for q in range(rows_per_tile // CB):
            pltpu.sync_copy(agg.at[pl.ds(r0 + q * CB, CB)], rows[0])
            pltpu.sync_copy(rows[0], out_h.at[pl.ds(r0 + q * CB, CB)])

    return mp_kernel


def _tc_norms_mm(deg_ref, x_ref, w1_ref, featn_ref, norms_ref):
    dsum = jnp.sum(deg_ref[...], axis=2)
    norms = jnp.where(dsum > 0, lax.rsqrt(jnp.maximum(dsum, 1e-12)), 0.0)
    norms_ref[...] = norms
    xw = jnp.dot(x_ref[...], w1_ref[...], preferred_element_type=jnp.float32)
    featn_ref[...] = xw * norms[:, 0:1]


def _tc_mid(agg_ref, norms_ref, b1_ref, out_ref):
    norms = norms_ref[...]
    h = agg_ref[...] * norms[:, 2:3] + b1_ref[...][None, :]
    h = jnp.maximum(h, 0.0)
    out_ref[...] = h * norms[:, 1:2]


def _tc_final(agg_ref, norms_ref, b2_ref, w2_ref, out_ref):
    f2 = jnp.dot(agg_ref[...], w2_ref[...], preferred_element_type=jnp.float32)
    z = f2 * norms_ref[...][:, 3:4] + b2_ref[...][None, :]
    m = jnp.max(z, axis=1, keepdims=True)
    ez = jnp.exp(z - m)
    out_ref[...] = ez / jnp.sum(ez, axis=1, keepdims=True)


def kernel(x, edge_index, edge_weight, W1, b1, W2, b2):
    n, d_in = x.shape
    e = edge_index.shape[1]
    d_hid = W1.shape[1]
    d_out = W2.shape[1]

    src = edge_index[0].astype(jnp.int32)
    dst = edge_index[1].astype(jnp.int32)

    np_ = _pad_n(n)
    n_blk = e // CB
    eidx = jnp.stack([src.reshape(n_blk, CB), dst.reshape(n_blk, CB)], axis=1)
    ew2d = edge_weight.reshape(n_blk, CB)

    n_blkp = ((n_blk + 2 * NW - 1) // (2 * NW)) * (2 * NW)
    e_pad = (n_blkp - n_blk) * CB
    pad_dst = (n + (jnp.arange(e_pad, dtype=jnp.int32) % (np_ - n))
               ).reshape(-1, CB)
    pad_idx = jnp.stack([jnp.zeros_like(pad_dst), pad_dst], axis=1)
    eidx_p = jnp.concatenate([eidx, pad_idx])
    ew2d_p = jnp.concatenate([ew2d, jnp.zeros((n_blkp - n_blk, CB),
                                              jnp.float32)])

    deg_part = _make_deg_kernel(n, n_blkp)(eidx_p, ew2d_p)
    deg = deg_part.reshape(NC, 4, np_)[:, :, :n].transpose(2, 1, 0)

    featn, norms = pl.pallas_call(
        _tc_norms_mm,
        out_shape=(
            jax.ShapeDtypeStruct((n, d_hid), jnp.float32),
            jax.ShapeDtypeStruct((n, 4), jnp.float32),
        ),
    )(deg, x, W1)

    agg1 = _make_mp_kernel(n, d_hid, n_blk, True)(featn, eidx, ew2d)[:n]

    hn = pl.pallas_call(
        _tc_mid,
        out_shape=jax.ShapeDtypeStruct((n, d_hid), jnp.float32),
    )(agg1, norms, b1)

    agg2 = _make_mp_kernel(n, d_hid, n_blk, False)(hn, eidx)[:n]

    return pl.pallas_call(
        _tc_final,
        out_shape=jax.ShapeDtypeStruct((n, d_out), jnp.float32),
    )(agg2, norms, b2, W2)
